# Initial kernel scaffold; baseline (speedup 1.0000x reference)
import jax
import jax.numpy as jnp
from jax.experimental import pallas as pl


def kernel(x, edge_index, edge_attr, W_emb1, b_emb1, W_emb2, b_emb2, M1, bM1, M2, bM2, U1, bU1, U2, bU2, Cw1, Cb1, Cw2, Cb2):
    raise NotImplementedError("write your pallas kernel here")


# speedup vs baseline: 1.8966x; 1.8966x over previous
import functools

import jax
import jax.numpy as jnp
from jax import lax
from jax.experimental import pallas as pl
from jax.experimental.pallas import tpu as pltpu
from jax.experimental.pallas import tpu_sc as plsc

N = 10000
E = 160000
H = 128
D_EDGE = 16
L = 4

NC = 2
NS = 16
NW = NC * NS
CH = 128
EPT = 5120
EPAD = NW * EPT
NCHUNK = EPT // CH
NP = 10112
RPT = NP // NS
_RCHUNKS = [(i * CH, CH) for i in range(RPT // CH)]
if RPT % CH:
    _RCHUNKS.append((RPT - RPT % CH, RPT % CH))
BE = 2048

_f32 = jnp.float32


def _mesh():
    return plsc.VectorSubcoreMesh(
        core_axis_name="c", subcore_axis_name="s", num_cores=NC, num_subcores=NS)


def _gather_body(p_hbm, q_hbm, dst_hbm, src_hbm, g_hbm,
                 didx, sidx, prows, qrows, sem1, sem2):
    c = lax.axis_index("c")
    s = lax.axis_index("s")
    wid = s * NC + c
    base = wid * EPT

    def chunk(i, carry):
        off = base + i * CH
        pltpu.sync_copy(dst_hbm.at[pl.ds(off, CH)], didx)
        pltpu.sync_copy(src_hbm.at[pl.ds(off, CH)], sidx)
        cp1 = pltpu.async_copy(p_hbm.at[didx], prows, sem1)
        cp2 = pltpu.async_copy(q_hbm.at[sidx], qrows, sem2)
        cp1.wait()
        cp2.wait()

        def row(r, carry2):
            for j in range(H // 16):
                sl = pl.ds(j * 16, 16)
                prows[r, sl] = prows[r, sl] + qrows[r, sl]
            return carry2

        lax.fori_loop(0, CH, row, 0)
        pltpu.sync_copy(prows, g_hbm.at[pl.ds(off, CH)])
        return carry

    lax.fori_loop(0, NCHUNK, chunk, 0)


def _gather_call(P, Q, dst_g, src_g):
    k = pl.kernel(
        _gather_body,
        out_type=jax.ShapeDtypeStruct((EPAD, H), _f32),
        mesh=_mesh(),
        scratch_types=[
            pltpu.VMEM((CH,), jnp.int32),
            pltpu.VMEM((CH,), jnp.int32),
            pltpu.VMEM((CH, H), _f32),
            pltpu.VMEM((CH, H), _f32),
            pltpu.SemaphoreType.DMA,
            pltpu.SemaphoreType.DMA,
        ],
    )
    return k(P, Q, dst_g, src_g)


def _zero_acc_stripes(cz_hbm, mrows, acc, s):
    pltpu.sync_copy(cz_hbm.at[0], mrows)
    for boff, bsz in _RCHUNKS:
        r0 = s * RPT + boff
        pltpu.sync_copy(mrows.at[pl.ds(0, bsz)], acc.at[pl.ds(r0, bsz)])


def _writeout_acc_stripes(parts_hbm, mrows, acc, c, s):
    for boff, bsz in _RCHUNKS:
        r0 = s * RPT + boff
        pltpu.sync_copy(acc.at[pl.ds(r0, bsz)], mrows.at[pl.ds(0, bsz)])
        pltpu.sync_copy(mrows.at[pl.ds(0, bsz)], parts_hbm.at[c, pl.ds(r0, bsz)])


def _scatter_body(m_hbm, dst_hbm, cz_hbm, parts_hbm, didx, mrows, acc, sem):
    del sem
    c = lax.axis_index("c")
    s = lax.axis_index("s")
    wid = s * NC + c
    _zero_acc_stripes(cz_hbm, mrows, acc, s)
    plsc.subcore_barrier()

    def chunk(i, carry):
        off = wid * EPT + i * CH
        pltpu.sync_copy(dst_hbm.at[pl.ds(off, CH)], didx)
        pltpu.sync_copy(m_hbm.at[pl.ds(off, CH)], mrows)
        pltpu.sync_copy(mrows, acc.at[didx], add=True)
        return carry

    lax.fori_loop(0, NCHUNK, chunk, 0)
    plsc.subcore_barrier()
    _writeout_acc_stripes(parts_hbm, mrows, acc, c, s)


def _counts_body(dst_hbm, cz_hbm, parts_hbm, didx, mrows, acc, sem):
    del sem
    c = lax.axis_index("c")
    s = lax.axis_index("s")
    wid = s * NC + c
    _zero_acc_stripes(cz_hbm, mrows, acc, s)
    plsc.subcore_barrier()
    pltpu.sync_copy(cz_hbm.at[1], mrows)

    def chunk(i, carry):
        off = wid * EPT + i * CH
        pltpu.sync_copy(dst_hbm.at[pl.ds(off, CH)], didx)
        pltpu.sync_copy(mrows, acc.at[didx], add=True)
        return carry

    lax.fori_loop(0, NCHUNK, chunk, 0)
    plsc.subcore_barrier()
    _writeout_acc_stripes(parts_hbm, mrows, acc, c, s)


_SC_SCRATCH = [
    pltpu.VMEM((CH,), jnp.int32),
    pltpu.VMEM((CH, H), _f32),
    pltpu.VMEM_SHARED((NP, H), _f32),
    pltpu.SemaphoreType.DMA,
]


def _scatter_call(m, dst_s, cz):
    k = pl.kernel(
        _scatter_body,
        out_type=jax.ShapeDtypeStruct((NC, NP, H), _f32),
        mesh=_mesh(),
        scratch_types=_SC_SCRATCH,
    )
    return k(m, dst_s, cz)


def _counts_call(dst_s, cz):
    k = pl.kernel(
        _counts_body,
        out_type=jax.ShapeDtypeStruct((NC, NP, H), _f32),
        mesh=_mesh(),
        scratch_types=_SC_SCRATCH,
    )
    return k(dst_s, cz)


def _embed_body(x_ref, w1_ref, b1_ref, w2_ref, b2_ref, ma_ref, mb_ref,
                h_ref, p_ref, q_ref):
    h = jnp.maximum(
        jnp.dot(x_ref[...], w1_ref[...], preferred_element_type=_f32)
        + b1_ref[...], 0.0)
    h = jnp.maximum(
        jnp.dot(h, w2_ref[...], preferred_element_type=_f32)
        + b2_ref[...], 0.0)
    h_ref[...] = h
    p_ref[...] = jnp.dot(h, ma_ref[...], preferred_element_type=_f32)
    q_ref[...] = jnp.dot(h, mb_ref[...], preferred_element_type=_f32)


def _embed_call(x, W1, b1, W2, b2, Ma, Mb):
    return pl.pallas_call(
        _embed_body,
        out_shape=(
            jax.ShapeDtypeStruct((N, H), _f32),
            jax.ShapeDtypeStruct((N, H), _f32),
            jax.ShapeDtypeStruct((N, H), _f32),
        ),
    )(x, W1, b1, W2, b2, Ma, Mb)


def _edge_body(g_ref, ea_ref, m1c_ref, bm1_ref, m2_ref, bm2_ref, o_ref):
    r = jnp.dot(ea_ref[...], m1c_ref[...], preferred_element_type=_f32) \
        + bm1_ref[...]
    t = jnp.maximum(g_ref[...] + r, 0.0)
    o_ref[...] = jnp.maximum(
        jnp.dot(t, m2_ref[...], preferred_element_type=_f32) + bm2_ref[...],
        0.0)


def _edge_call(g, ea_pad, M1c, bM1l, M2l, bM2l):
    return pl.pallas_call(
        _edge_body,
        grid=(EPAD // BE,),
        in_specs=[
            pl.BlockSpec((BE, H), lambda i: (i, 0)),
            pl.BlockSpec((BE, D_EDGE), lambda i: (i, 0)),
            pl.BlockSpec((D_EDGE, H), lambda i: (0, 0)),
            pl.BlockSpec((1, H), lambda i: (0, 0)),
            pl.BlockSpec((H, H), lambda i: (0, 0)),
            pl.BlockSpec((1, H), lambda i: (0, 0)),
        ],
        out_specs=pl.BlockSpec((BE, H), lambda i: (i, 0)),
        out_shape=jax.ShapeDtypeStruct((EPAD, H), _f32),
    )(g, ea_pad, M1c, bM1l, M2l, bM2l)


def _agg_from_parts(parts_ref, cparts_ref):
    counts = cparts_ref[0, :N, 0] + cparts_ref[1, :N, 0]
    inv = 1.0 / jnp.maximum(counts, 1.0)
    agg = (parts_ref[0, :N, :] + parts_ref[1, :N, :]) * inv[:, None]
    return agg


def _node_mlp(h, agg, u1a_ref, u1b_ref, bu1_ref, u2_ref, bu2_ref):
    u = jnp.maximum(
        jnp.dot(h, u1a_ref[...], preferred_element_type=_f32)
        + jnp.dot(agg, u1b_ref[...], preferred_element_type=_f32)
        + bu1_ref[...], 0.0)
    u = jnp.maximum(
        jnp.dot(u, u2_ref[...], preferred_element_type=_f32)
        + bu2_ref[...], 0.0)
    return h + u


def _node_body(h_ref, parts_ref, cparts_ref, u1a_ref, u1b_ref, bu1_ref,
               u2_ref, bu2_ref, ma_ref, mb_ref, h_out, p_out, q_out):
    agg = _agg_from_parts(parts_ref, cparts_ref)
    hn = _node_mlp(h_ref[...], agg, u1a_ref, u1b_ref, bu1_ref, u2_ref, bu2_ref)
    h_out[...] = hn
    p_out[...] = jnp.dot(hn, ma_ref[...], preferred_element_type=_f32)
    q_out[...] = jnp.dot(hn, mb_ref[...], preferred_element_type=_f32)


def _node_call(h, parts, cparts, U1a, U1b, bU1l, U2l, bU2l, Ma, Mb):
    return pl.pallas_call(
        _node_body,
        out_shape=(
            jax.ShapeDtypeStruct((N, H), _f32),
            jax.ShapeDtypeStruct((N, H), _f32),
            jax.ShapeDtypeStruct((N, H), _f32),
        ),
    )(h, parts, cparts, U1a, U1b, bU1l, U2l, bU2l, Ma, Mb)


def _final_body(h_ref, parts_ref, cparts_ref, u1a_ref, u1b_ref, bu1_ref,
                u2_ref, bu2_ref, a_ref, ba_ref, b_ref, bb_ref, o_ref):
    agg = _agg_from_parts(parts_ref, cparts_ref)
    hn = _node_mlp(h_ref[...], agg, u1a_ref, u1b_ref, bu1_ref, u2_ref, bu2_ref)
    z = jnp.maximum(
        jnp.dot(hn, a_ref[...], preferred_element_type=_f32) + ba_ref[...],
        0.0)
    o_ref[...] = jnp.dot(z, b_ref[...], preferred_element_type=_f32) \
        + bb_ref[...]


def _final_call(h, parts, cparts, U1a, U1b, bU1l, U2l, bU2l, A, bA, B, bB):
    return pl.pallas_call(
        _final_body,
        out_shape=jax.ShapeDtypeStruct((N, 20), _f32),
    )(h, parts, cparts, U1a, U1b, bU1l, U2l, bU2l, A, bA, B, bB)


def _build_decoder_mats(Cw1, Cb1, Cw2, Cb2):
    k = jnp.arange(15)
    t = jnp.arange(29)
    rows = 4 * t[None, :] + k[:, None]
    tcol = jnp.broadcast_to(t[None, :], (15, 29))
    valsA = jnp.broadcast_to(Cw1[:, 0, :].T[:, None, :], (15, 29, 8))
    A = jnp.zeros((128, 8, 29), _f32).at[rows, :, tcol].set(
        jnp.transpose(valsA, (0, 1, 2)))
    A = A.reshape(128, 8 * 29)
    bA = jnp.broadcast_to(Cb1[:, None], (8, 29)).reshape(1, 8 * 29)
    dt = jnp.arange(10)
    tp = jnp.arange(20)
    tt = tp[None, :] + dt[:, None]
    tpb = jnp.broadcast_to(tp[None, :], (10, 20))
    valsB = jnp.broadcast_to(Cw2[0][:, :, None], (8, 10, 20))
    B = jnp.zeros((8, 29, 20), _f32).at[:, tt, tpb].set(valsB)
    B = B.reshape(8 * 29, 20)
    bB = Cw2.dtype.type(0) + Cb2.reshape(1, 1)
    return A, bA, B, bB


def kernel(x, edge_index, edge_attr, W_emb1, b_emb1, W_emb2, b_emb2,
           M1, bM1, M2, bM2, U1, bU1, U2, bU2, Cw1, Cb1, Cw2, Cb2):
    pad = EPAD - E
    src = edge_index[0]
    dst = edge_index[1]
    izeros = jnp.zeros((pad,), jnp.int32)
    dst_g = jnp.concatenate([dst, izeros])
    src_g = jnp.concatenate([src, izeros])
    dst_s = jnp.concatenate([dst, jnp.full((pad,), NP - 1, jnp.int32)])
    ea_pad = jnp.concatenate(
        [edge_attr, jnp.zeros((pad, D_EDGE), _f32)], axis=0)
    cz = jnp.stack([jnp.zeros((CH, H), _f32), jnp.ones((CH, H), _f32)])
    cnt8 = _counts_call(dst_s, cz)[:, :, :8]

    h, P, Q = _embed_call(
        x, W_emb1[:H], b_emb1.reshape(1, H), W_emb2, b_emb2.reshape(1, H),
        M1[0, :H], M1[0, H:2 * H])

    out = None
    for l in range(L):
        g = _gather_call(P, Q, dst_g, src_g)
        m = _edge_call(g, ea_pad, M1[l, 2 * H:], bM1[l].reshape(1, H),
                       M2[l], bM2[l].reshape(1, H))
        parts = _scatter_call(m, dst_s, cz)
        if l < L - 1:
            h, P, Q = _node_call(
                h, parts, cnt8, U1[l, :H], U1[l, H:], bU1[l].reshape(1, H),
                U2[l], bU2[l].reshape(1, H), M1[l + 1, :H], M1[l + 1, H:2 * H])
        else:
            A, bA, B, bB = _build_decoder_mats(Cw1, Cb1, Cw2, Cb2)
            out = _final_call(
                h, parts, cnt8, U1[l, :H], U1[l, H:], bU1[l].reshape(1, H),
                U2[l], bU2[l].reshape(1, H), A, bA, B, bB)
    return out


# speedup vs baseline: 2.1755x; 1.1470x over previous
import functools

import jax
import jax.numpy as jnp
from jax import lax
from jax.experimental import pallas as pl
from jax.experimental.pallas import tpu as pltpu
from jax.experimental.pallas import tpu_sc as plsc

N = 10000
E = 160000
H = 128
D_EDGE = 16
L = 4

NC = 2
NS = 16
NW = NC * NS
CH = 128
EPT = 5120
EPAD = NW * EPT
NCHUNK = EPT // CH
NP = 10112
RPT = NP // NS
_RCHUNKS = [(i * CH, CH) for i in range(RPT // CH)]
if RPT % CH:
    _RCHUNKS.append((RPT - RPT % CH, RPT % CH))
BE = 2048

_f32 = jnp.float32


def _mesh():
    return plsc.VectorSubcoreMesh(
        core_axis_name="c", subcore_axis_name="s", num_cores=NC, num_subcores=NS)


def _drain(src, dst, sem):
    pltpu.make_async_copy(src, dst, sem).wait()


def _gather_body(p_hbm, q_hbm, dst_hbm, src_hbm, g_hbm,
                 dall, sall, pb0, pb1, qb0, qb1,
                 sp0, sp1, sq0, sq1, wp0, wp1, wq0, wq1):
    c = lax.axis_index("c")
    s = lax.axis_index("s")
    wid = s * NC + c
    base = wid * EPT
    pbuf = (pb0, pb1)
    qbuf = (qb0, qb1)
    sp = (sp0, sp1)
    sq = (sq0, sq1)
    wp = (wp0, wp1)
    wq = (wq0, wq1)

    pltpu.sync_copy(dst_hbm.at[pl.ds(base, EPT)], dall)
    pltpu.sync_copy(src_hbm.at[pl.ds(base, EPT)], sall)

    def body(k, carry):
        for b in (0, 1):
            i = 2 * k + b

            @pl.when(k > 0)
            def _free():
                _drain(pbuf[b], g_hbm.at[pl.ds(base, CH), pl.ds(0, H)], wp[b])
                _drain(qbuf[b], g_hbm.at[pl.ds(base, CH), pl.ds(H, H)], wq[b])

            pltpu.async_copy(p_hbm.at[dall.at[pl.ds(i * CH, CH)]],
                             pbuf[b], sp[b])
            pltpu.async_copy(q_hbm.at[sall.at[pl.ds(i * CH, CH)]],
                             qbuf[b], sq[b])
        for b in (0, 1):
            i = 2 * k + b
            off = base + i * CH
            _drain(p_hbm.at[pl.ds(0, CH)], pbuf[b], sp[b])
            _drain(q_hbm.at[pl.ds(0, CH)], qbuf[b], sq[b])
            pltpu.async_copy(pbuf[b], g_hbm.at[pl.ds(off, CH), pl.ds(0, H)],
                             wp[b])
            pltpu.async_copy(qbuf[b], g_hbm.at[pl.ds(off, CH), pl.ds(H, H)],
                             wq[b])
        return carry

    lax.fori_loop(0, NCHUNK // 2, body, 0)
    for b in (0, 1):
        _drain(pbuf[b], g_hbm.at[pl.ds(base, CH), pl.ds(0, H)], wp[b])
        _drain(qbuf[b], g_hbm.at[pl.ds(base, CH), pl.ds(H, H)], wq[b])


def _gather_call(P, Q, dst_g, src_g):
    k = pl.kernel(
        _gather_body,
        out_type=jax.ShapeDtypeStruct((EPAD, 2 * H), _f32),
        mesh=_mesh(),
        scratch_types=[
            pltpu.VMEM((EPT,), jnp.int32),
            pltpu.VMEM((EPT,), jnp.int32),
            pltpu.VMEM((CH, H), _f32),
            pltpu.VMEM((CH, H), _f32),
            pltpu.VMEM((CH, H), _f32),
            pltpu.VMEM((CH, H), _f32),
        ] + [pltpu.SemaphoreType.DMA] * 8,
    )
    return k(P, Q, dst_g, src_g)


def _zero_acc_stripes(cz_hbm, mrows, acc, s):
    pltpu.sync_copy(cz_hbm.at[0], mrows)
    for boff, bsz in _RCHUNKS:
        r0 = s * RPT + boff
        pltpu.sync_copy(mrows.at[pl.ds(0, bsz)], acc.at[pl.ds(r0, bsz)])


def _writeout_acc_stripes(parts_hbm, mrows, acc, c, s):
    for boff, bsz in _RCHUNKS:
        r0 = s * RPT + boff
        pltpu.sync_copy(acc.at[pl.ds(r0, bsz)], mrows.at[pl.ds(0, bsz)])
        pltpu.sync_copy(mrows.at[pl.ds(0, bsz)], parts_hbm.at[c, pl.ds(r0, bsz)])


def _scatter_body(m_hbm, dst3_hbm, cz_hbm, parts_hbm,
                  didx2, mb0, mb1, acc, sm0, sm1, sc0, sc1):
    c = lax.axis_index("c")
    s = lax.axis_index("s")
    wid = s * NC + c
    mbuf = (mb0, mb1)
    sm = (sm0, sm1)
    sc = (sc0, sc1)
    _zero_acc_stripes(cz_hbm, mb0, acc, s)
    pltpu.sync_copy(dst3_hbm.at[wid], didx2)
    plsc.subcore_barrier()

    def body(k, carry):
        for b in (0, 1):
            i = 2 * k + b

            @pl.when(k > 0)
            def _free():
                _drain(mbuf[b], acc.at[didx2.at[0]], sc[b])

            off = wid * EPT + i * CH
            pltpu.async_copy(m_hbm.at[pl.ds(off, CH)], mbuf[b], sm[b])
        for b in (0, 1):
            i = 2 * k + b
            _drain(m_hbm.at[pl.ds(0, CH)], mbuf[b], sm[b])
            pltpu.async_copy(mbuf[b], acc.at[didx2.at[i]], sc[b], add=True)
        return carry

    lax.fori_loop(0, NCHUNK // 2, body, 0)
    for b in (0, 1):
        _drain(mbuf[b], acc.at[didx2.at[0]], sc[b])
    plsc.subcore_barrier()
    _writeout_acc_stripes(parts_hbm, mb0, acc, c, s)


def _counts_body(dst3_hbm, cz_hbm, parts_hbm,
                 didx2, mb0, mb1, acc, sm0, sm1, sc0, sc1):
    del sm0, sm1
    c = lax.axis_index("c")
    s = lax.axis_index("s")
    wid = s * NC + c
    sc = (sc0, sc1)
    _zero_acc_stripes(cz_hbm, mb0, acc, s)
    pltpu.sync_copy(dst3_hbm.at[wid], didx2)
    pltpu.sync_copy(cz_hbm.at[1], mb1)
    plsc.subcore_barrier()

    def body(k, carry):
        for b in (0, 1):
            i = 2 * k + b

            @pl.when(k > 0)
            def _free():
                _drain(mb1, acc.at[didx2.at[0]], sc[b])

            pltpu.async_copy(mb1, acc.at[didx2.at[i]], sc[b], add=True)
        return carry

    lax.fori_loop(0, NCHUNK // 2, body, 0)
    for b in (0, 1):
        _drain(mb1, acc.at[didx2.at[0]], sc[b])
    plsc.subcore_barrier()
    _writeout_acc_stripes(parts_hbm, mb0, acc, c, s)


_SC_SCRATCH = [
    pltpu.VMEM((NCHUNK, CH), jnp.int32),
    pltpu.VMEM((CH, H), _f32),
    pltpu.VMEM((CH, H), _f32),
    pltpu.VMEM_SHARED((NP, H), _f32),
] + [pltpu.SemaphoreType.DMA] * 4


def _scatter_call(m, dst3, cz):
    k = pl.kernel(
        _scatter_body,
        out_type=jax.ShapeDtypeStruct((NC, NP, H), _f32),
        mesh=_mesh(),
        scratch_types=_SC_SCRATCH,
    )
    return k(m, dst3, cz)


def _counts_call(dst3, cz):
    k = pl.kernel(
        _counts_body,
        out_type=jax.ShapeDtypeStruct((NC, NP, H), _f32),
        mesh=_mesh(),
        scratch_types=_SC_SCRATCH,
    )
    return k(dst3, cz)


def _embed_body(x_ref, w1_ref, b1_ref, w2_ref, b2_ref, ma_ref, mb_ref,
                h_ref, p_ref, q_ref):
    h = jnp.maximum(
        jnp.dot(x_ref[...], w1_ref[...], preferred_element_type=_f32)
        + b1_ref[...], 0.0)
    h = jnp.maximum(
        jnp.dot(h, w2_ref[...], preferred_element_type=_f32)
        + b2_ref[...], 0.0)
    h_ref[...] = h
    p_ref[...] = jnp.dot(h, ma_ref[...], preferred_element_type=_f32)
    q_ref[...] = jnp.dot(h, mb_ref[...], preferred_element_type=_f32)


def _embed_call(x, W1, b1, W2, b2, Ma, Mb):
    return pl.pallas_call(
        _embed_body,
        out_shape=(
            jax.ShapeDtypeStruct((N, H), _f32),
            jax.ShapeDtypeStruct((N, H), _f32),
            jax.ShapeDtypeStruct((N, H), _f32),
        ),
    )(x, W1, b1, W2, b2, Ma, Mb)


def _edge_body(g_ref, ea_ref, m1c_ref, bm1_ref, m2_ref, bm2_ref, o_ref):
    r = jnp.dot(ea_ref[...], m1c_ref[...], preferred_element_type=_f32) \
        + bm1_ref[...]
    t = jnp.maximum(g_ref[:, :H] + g_ref[:, H:] + r, 0.0)
    o_ref[...] = jnp.maximum(
        jnp.dot(t, m2_ref[...], preferred_element_type=_f32) + bm2_ref[...],
        0.0)


def _edge_call(g, ea_pad, M1c, bM1l, M2l, bM2l):
    return pl.pallas_call(
        _edge_body,
        grid=(EPAD // BE,),
        in_specs=[
            pl.BlockSpec((BE, 2 * H), lambda i: (i, 0)),
            pl.BlockSpec((BE, D_EDGE), lambda i: (i, 0)),
            pl.BlockSpec((D_EDGE, H), lambda i: (0, 0)),
            pl.BlockSpec((1, H), lambda i: (0, 0)),
            pl.BlockSpec((H, H), lambda i: (0, 0)),
            pl.BlockSpec((1, H), lambda i: (0, 0)),
        ],
        out_specs=pl.BlockSpec((BE, H), lambda i: (i, 0)),
        out_shape=jax.ShapeDtypeStruct((EPAD, H), _f32),
    )(g, ea_pad, M1c, bM1l, M2l, bM2l)


def _agg_from_parts(parts_ref, cparts_ref):
    counts = cparts_ref[0, :N, 0] + cparts_ref[1, :N, 0]
    inv = 1.0 / jnp.maximum(counts, 1.0)
    agg = (parts_ref[0, :N, :] + parts_ref[1, :N, :]) * inv[:, None]
    return agg


def _node_mlp(h, agg, u1a_ref, u1b_ref, bu1_ref, u2_ref, bu2_ref):
    u = jnp.maximum(
        jnp.dot(h, u1a_ref[...], preferred_element_type=_f32)
        + jnp.dot(agg, u1b_ref[...], preferred_element_type=_f32)
        + bu1_ref[...], 0.0)
    u = jnp.maximum(
        jnp.dot(u, u2_ref[...], preferred_element_type=_f32)
        + bu2_ref[...], 0.0)
    return h + u


def _node_body(h_ref, parts_ref, cparts_ref, u1a_ref, u1b_ref, bu1_ref,
               u2_ref, bu2_ref, ma_ref, mb_ref, h_out, p_out, q_out):
    agg = _agg_from_parts(parts_ref, cparts_ref)
    hn = _node_mlp(h_ref[...], agg, u1a_ref, u1b_ref, bu1_ref, u2_ref, bu2_ref)
    h_out[...] = hn
    p_out[...] = jnp.dot(hn, ma_ref[...], preferred_element_type=_f32)
    q_out[...] = jnp.dot(hn, mb_ref[...], preferred_element_type=_f32)


def _node_call(h, parts, cparts, U1a, U1b, bU1l, U2l, bU2l, Ma, Mb):
    return pl.pallas_call(
        _node_body,
        out_shape=(
            jax.ShapeDtypeStruct((N, H), _f32),
            jax.ShapeDtypeStruct((N, H), _f32),
            jax.ShapeDtypeStruct((N, H), _f32),
        ),
    )(h, parts, cparts, U1a, U1b, bU1l, U2l, bU2l, Ma, Mb)


def _final_body(h_ref, parts_ref, cparts_ref, u1a_ref, u1b_ref, bu1_ref,
                u2_ref, bu2_ref, a_ref, ba_ref, b_ref, bb_ref, o_ref):
    agg = _agg_from_parts(parts_ref, cparts_ref)
    hn = _node_mlp(h_ref[...], agg, u1a_ref, u1b_ref, bu1_ref, u2_ref, bu2_ref)
    z = jnp.maximum(
        jnp.dot(hn, a_ref[...], preferred_element_type=_f32) + ba_ref[...],
        0.0)
    o_ref[...] = jnp.dot(z, b_ref[...], preferred_element_type=_f32) \
        + bb_ref[...]


def _final_call(h, parts, cparts, U1a, U1b, bU1l, U2l, bU2l, A, bA, B, bB):
    return pl.pallas_call(
        _final_body,
        out_shape=jax.ShapeDtypeStruct((N, 20), _f32),
    )(h, parts, cparts, U1a, U1b, bU1l, U2l, bU2l, A, bA, B, bB)


def _build_decoder_mats(Cw1, Cb1, Cw2, Cb2):
    k = jnp.arange(15)
    t = jnp.arange(29)
    rows = 4 * t[None, :] + k[:, None]
    tcol = jnp.broadcast_to(t[None, :], (15, 29))
    valsA = jnp.broadcast_to(Cw1[:, 0, :].T[:, None, :], (15, 29, 8))
    A = jnp.zeros((128, 8, 29), _f32).at[rows, :, tcol].set(
        jnp.transpose(valsA, (0, 1, 2)))
    A = A.reshape(128, 8 * 29)
    bA = jnp.broadcast_to(Cb1[:, None], (8, 29)).reshape(1, 8 * 29)
    dt = jnp.arange(10)
    tp = jnp.arange(20)
    tt = tp[None, :] + dt[:, None]
    tpb = jnp.broadcast_to(tp[None, :], (10, 20))
    valsB = jnp.broadcast_to(Cw2[0][:, :, None], (8, 10, 20))
    B = jnp.zeros((8, 29, 20), _f32).at[:, tt, tpb].set(valsB)
    B = B.reshape(8 * 29, 20)
    bB = Cw2.dtype.type(0) + Cb2.reshape(1, 1)
    return A, bA, B, bB


def kernel(x, edge_index, edge_attr, W_emb1, b_emb1, W_emb2, b_emb2,
           M1, bM1, M2, bM2, U1, bU1, U2, bU2, Cw1, Cb1, Cw2, Cb2):
    pad = EPAD - E
    src = edge_index[0]
    dst = edge_index[1]
    izeros = jnp.zeros((pad,), jnp.int32)
    dst_g = jnp.concatenate([dst, izeros])
    src_g = jnp.concatenate([src, izeros])
    dst_s = jnp.concatenate([dst, jnp.full((pad,), NP - 1, jnp.int32)])
    dst3 = dst_s.reshape(NW, NCHUNK, CH)
    ea_pad = jnp.concatenate(
        [edge_attr, jnp.zeros((pad, D_EDGE), _f32)], axis=0)
    cz = jnp.stack([jnp.zeros((CH, H), _f32), jnp.ones((CH, H), _f32)])
    cnt8 = _counts_call(dst3, cz)[:, :, :8]

    h, P, Q = _embed_call(
        x, W_emb1[:H], b_emb1.reshape(1, H), W_emb2, b_emb2.reshape(1, H),
        M1[0, :H], M1[0, H:2 * H])

    out = None
    for l in range(L):
        g = _gather_call(P, Q, dst_g, src_g)
        m = _edge_call(g, ea_pad, M1[l, 2 * H:], bM1[l].reshape(1, H),
                       M2[l], bM2[l].reshape(1, H))
        parts = _scatter_call(m, dst3, cz)
        if l < L - 1:
            h, P, Q = _node_call(
                h, parts, cnt8, U1[l, :H], U1[l, H:], bU1[l].reshape(1, H),
                U2[l], bU2[l].reshape(1, H), M1[l + 1, :H], M1[l + 1, H:2 * H])
        else:
            A, bA, B, bB = _build_decoder_mats(Cw1, Cb1, Cw2, Cb2)
            out = _final_call(
                h, parts, cnt8, U1[l, :H], U1[l, H:], bU1[l].reshape(1, H),
                U2[l], bU2[l].reshape(1, H), A, bA, B, bB)
    return out


# speedup vs baseline: 2.2966x; 1.0557x over previous
import functools

import jax
import jax.numpy as jnp
from jax import lax
from jax.experimental import pallas as pl
from jax.experimental.pallas import tpu as pltpu
from jax.experimental.pallas import tpu_sc as plsc

N = 10000
E = 160000
H = 128
D_EDGE = 16
L = 4

NC = 2
NS = 16
NW = NC * NS
CH = 80
RING = 4
EPT = 5120
EPAD = NW * EPT
NCHUNK = EPT // CH
NP = 10112
RPT = NP // NS
_RCHUNKS = [(i * CH, CH) for i in range(RPT // CH)]
if RPT % CH:
    _RCHUNKS.append((RPT - RPT % CH, RPT % CH))
BE = 2048

_f32 = jnp.float32


def _mesh():
    return plsc.VectorSubcoreMesh(
        core_axis_name="c", subcore_axis_name="s", num_cores=NC, num_subcores=NS)


def _drain(src, dst, sem):
    pltpu.make_async_copy(src, dst, sem).wait()


def _gather_body(p_hbm, q_hbm, dst_hbm, src_hbm, g_hbm, dall, sall, *bufs):
    c = lax.axis_index("c")
    s = lax.axis_index("s")
    wid = s * NC + c
    base = wid * EPT
    pbuf = bufs[0:RING]
    qbuf = bufs[RING:2 * RING]
    sp = bufs[2 * RING:3 * RING]
    sq = bufs[3 * RING:4 * RING]
    wp = bufs[4 * RING:5 * RING]
    wq = bufs[5 * RING:6 * RING]

    pltpu.sync_copy(dst_hbm.at[pl.ds(base, EPT)], dall)
    pltpu.sync_copy(src_hbm.at[pl.ds(base, EPT)], sall)

    def issue_gather(i, b):
        pltpu.async_copy(p_hbm.at[dall.at[pl.ds(i * CH, CH)]], pbuf[b], sp[b])
        pltpu.async_copy(q_hbm.at[sall.at[pl.ds(i * CH, CH)]], qbuf[b], sq[b])

    def issue_writeout(j, b):
        _drain(p_hbm.at[pl.ds(0, CH)], pbuf[b], sp[b])
        _drain(q_hbm.at[pl.ds(0, CH)], qbuf[b], sq[b])
        off = base + j * CH
        pltpu.async_copy(pbuf[b], g_hbm.at[pl.ds(off, CH), pl.ds(0, H)], wp[b])
        pltpu.async_copy(qbuf[b], g_hbm.at[pl.ds(off, CH), pl.ds(H, H)], wq[b])

    def drain_writeout(b):
        _drain(pbuf[b], g_hbm.at[pl.ds(base, CH), pl.ds(0, H)], wp[b])
        _drain(qbuf[b], g_hbm.at[pl.ds(base, CH), pl.ds(H, H)], wq[b])

    def body(k, carry):
        for b in range(RING):
            i = RING * k + b

            @pl.when(k > 0)
            def _free():
                drain_writeout(b)

            issue_gather(i, b)
            bb = (b + 1) % RING
            j = i - (RING - 1)
            if b == RING - 1:
                issue_writeout(RING * k, bb)
            else:
                @pl.when(k > 0)
                def _wout():
                    issue_writeout(j, bb)
        return carry

    lax.fori_loop(0, NCHUNK // RING, body, 0)
    for t in range(RING - 1):
        j = NCHUNK - (RING - 1) + t
        issue_writeout(j, j % RING)
    for b in range(RING):
        drain_writeout(b)


def _gather_call(P, Q, dst_g, src_g):
    k = pl.kernel(
        _gather_body,
        out_type=jax.ShapeDtypeStruct((EPAD, 2 * H), _f32),
        mesh=_mesh(),
        scratch_types=[
            pltpu.VMEM((EPT,), jnp.int32),
            pltpu.VMEM((EPT,), jnp.int32),
        ] + [pltpu.VMEM((CH, H), _f32)] * (2 * RING)
          + [pltpu.SemaphoreType.DMA] * (4 * RING),
    )
    return k(P, Q, dst_g, src_g)


def _zero_acc_stripes(cz_hbm, mrows, acc, s):
    pltpu.sync_copy(cz_hbm.at[0], mrows)
    for boff, bsz in _RCHUNKS:
        r0 = s * RPT + boff
        pltpu.sync_copy(mrows.at[pl.ds(0, bsz)], acc.at[pl.ds(r0, bsz)])


def _writeout_acc_stripes(parts_hbm, mrows, acc, c, s):
    for boff, bsz in _RCHUNKS:
        r0 = s * RPT + boff
        pltpu.sync_copy(acc.at[pl.ds(r0, bsz)], mrows.at[pl.ds(0, bsz)])
        pltpu.sync_copy(mrows.at[pl.ds(0, bsz)], parts_hbm.at[c, pl.ds(r0, bsz)])


def _scatter_body(m_hbm, dst3_hbm, cz_hbm, parts_hbm, didx2, acc, *bufs):
    c = lax.axis_index("c")
    s = lax.axis_index("s")
    wid = s * NC + c
    mbuf = bufs[0:RING]
    sm = bufs[RING:2 * RING]
    sc = bufs[2 * RING:3 * RING]
    _zero_acc_stripes(cz_hbm, mbuf[0], acc, s)
    pltpu.sync_copy(dst3_hbm.at[wid], didx2)
    plsc.subcore_barrier()

    def issue_load(i, b):
        off = wid * EPT + i * CH
        pltpu.async_copy(m_hbm.at[pl.ds(off, CH)], mbuf[b], sm[b])

    def issue_scatter(j, b):
        _drain(m_hbm.at[pl.ds(0, CH)], mbuf[b], sm[b])
        pltpu.async_copy(mbuf[b], acc.at[didx2.at[j]], sc[b], add=True)

    def drain_scatter(b):
        _drain(mbuf[b], acc.at[didx2.at[0]], sc[b])

    def body(k, carry):
        for b in range(RING):
            i = RING * k + b

            @pl.when(k > 0)
            def _free():
                drain_scatter(b)

            issue_load(i, b)
            bb = (b + 1) % RING
            j = i - (RING - 1)
            if b == RING - 1:
                issue_scatter(RING * k, bb)
            else:
                @pl.when(k > 0)
                def _sc():
                    issue_scatter(j, bb)
        return carry

    lax.fori_loop(0, NCHUNK // RING, body, 0)
    for t in range(RING - 1):
        j = NCHUNK - (RING - 1) + t
        issue_scatter(j, j % RING)
    for b in range(RING):
        drain_scatter(b)
    plsc.subcore_barrier()
    _writeout_acc_stripes(parts_hbm, mbuf[0], acc, c, s)


def _counts_body(dst3_hbm, cz_hbm, parts_hbm, didx2, acc, *bufs):
    c = lax.axis_index("c")
    s = lax.axis_index("s")
    wid = s * NC + c
    zb = bufs[0]
    ones = bufs[1]
    sc = bufs[2 * RING:3 * RING]
    _zero_acc_stripes(cz_hbm, zb, acc, s)
    pltpu.sync_copy(dst3_hbm.at[wid], didx2)
    pltpu.sync_copy(cz_hbm.at[1], ones)
    plsc.subcore_barrier()

    def body(k, carry):
        for b in range(RING):
            i = RING * k + b

            @pl.when(k > 0)
            def _free():
                _drain(ones, acc.at[didx2.at[0]], sc[b])

            pltpu.async_copy(ones, acc.at[didx2.at[i]], sc[b], add=True)
        return carry

    lax.fori_loop(0, NCHUNK // RING, body, 0)
    for b in range(RING):
        _drain(ones, acc.at[didx2.at[0]], sc[b])
    plsc.subcore_barrier()
    _writeout_acc_stripes(parts_hbm, zb, acc, c, s)


_SC_SCRATCH = [
    pltpu.VMEM((NCHUNK, CH), jnp.int32),
    pltpu.VMEM_SHARED((NP, H), _f32),
] + [pltpu.VMEM((CH, H), _f32)] * RING \
  + [pltpu.SemaphoreType.DMA] * (2 * RING)


def _scatter_call(m, dst3, cz):
    k = pl.kernel(
        _scatter_body,
        out_type=jax.ShapeDtypeStruct((NC, NP, H), _f32),
        mesh=_mesh(),
        scratch_types=_SC_SCRATCH,
    )
    return k(m, dst3, cz)


def _counts_call(dst3, cz):
    k = pl.kernel(
        _counts_body,
        out_type=jax.ShapeDtypeStruct((NC, NP, H), _f32),
        mesh=_mesh(),
        scratch_types=_SC_SCRATCH,
    )
    return k(dst3, cz)


def _embed_body(x_ref, w1_ref, b1_ref, w2_ref, b2_ref, ma_ref, mb_ref,
                h_ref, p_ref, q_ref):
    h = jnp.maximum(
        jnp.dot(x_ref[...], w1_ref[...], preferred_element_type=_f32)
        + b1_ref[...], 0.0)
    h = jnp.maximum(
        jnp.dot(h, w2_ref[...], preferred_element_type=_f32)
        + b2_ref[...], 0.0)
    h_ref[...] = h
    p_ref[...] = jnp.dot(h, ma_ref[...], preferred_element_type=_f32)
    q_ref[...] = jnp.dot(h, mb_ref[...], preferred_element_type=_f32)


def _embed_call(x, W1, b1, W2, b2, Ma, Mb):
    return pl.pallas_call(
        _embed_body,
        out_shape=(
            jax.ShapeDtypeStruct((N, H), _f32),
            jax.ShapeDtypeStruct((N, H), _f32),
            jax.ShapeDtypeStruct((N, H), _f32),
        ),
    )(x, W1, b1, W2, b2, Ma, Mb)


def _edge_body(g_ref, ea_ref, m1c_ref, bm1_ref, m2_ref, bm2_ref, o_ref):
    r = jnp.dot(ea_ref[...], m1c_ref[...], preferred_element_type=_f32) \
        + bm1_ref[...]
    t = jnp.maximum(g_ref[:, :H] + g_ref[:, H:] + r, 0.0)
    o_ref[...] = jnp.maximum(
        jnp.dot(t, m2_ref[...], preferred_element_type=_f32) + bm2_ref[...],
        0.0)


def _edge_call(g, ea_pad, M1c, bM1l, M2l, bM2l):
    return pl.pallas_call(
        _edge_body,
        grid=(EPAD // BE,),
        in_specs=[
            pl.BlockSpec((BE, 2 * H), lambda i: (i, 0)),
            pl.BlockSpec((BE, D_EDGE), lambda i: (i, 0)),
            pl.BlockSpec((D_EDGE, H), lambda i: (0, 0)),
            pl.BlockSpec((1, H), lambda i: (0, 0)),
            pl.BlockSpec((H, H), lambda i: (0, 0)),
            pl.BlockSpec((1, H), lambda i: (0, 0)),
        ],
        out_specs=pl.BlockSpec((BE, H), lambda i: (i, 0)),
        out_shape=jax.ShapeDtypeStruct((EPAD, H), _f32),
    )(g, ea_pad, M1c, bM1l, M2l, bM2l)


def _agg_from_parts(parts_ref, cparts_ref):
    counts = cparts_ref[0, :N, 0] + cparts_ref[1, :N, 0]
    inv = 1.0 / jnp.maximum(counts, 1.0)
    agg = (parts_ref[0, :N, :] + parts_ref[1, :N, :]) * inv[:, None]
    return agg


def _node_mlp(h, agg, u1a_ref, u1b_ref, bu1_ref, u2_ref, bu2_ref):
    u = jnp.maximum(
        jnp.dot(h, u1a_ref[...], preferred_element_type=_f32)
        + jnp.dot(agg, u1b_ref[...], preferred_element_type=_f32)
        + bu1_ref[...], 0.0)
    u = jnp.maximum(
        jnp.dot(u, u2_ref[...], preferred_element_type=_f32)
        + bu2_ref[...], 0.0)
    return h + u


def _node_body(h_ref, parts_ref, cparts_ref, u1a_ref, u1b_ref, bu1_ref,
               u2_ref, bu2_ref, ma_ref, mb_ref, h_out, p_out, q_out):
    agg = _agg_from_parts(parts_ref, cparts_ref)
    hn = _node_mlp(h_ref[...], agg, u1a_ref, u1b_ref, bu1_ref, u2_ref, bu2_ref)
    h_out[...] = hn
    p_out[...] = jnp.dot(hn, ma_ref[...], preferred_element_type=_f32)
    q_out[...] = jnp.dot(hn, mb_ref[...], preferred_element_type=_f32)


def _node_call(h, parts, cparts, U1a, U1b, bU1l, U2l, bU2l, Ma, Mb):
    return pl.pallas_call(
        _node_body,
        out_shape=(
            jax.ShapeDtypeStruct((N, H), _f32),
            jax.ShapeDtypeStruct((N, H), _f32),
            jax.ShapeDtypeStruct((N, H), _f32),
        ),
    )(h, parts, cparts, U1a, U1b, bU1l, U2l, bU2l, Ma, Mb)


def _final_body(h_ref, parts_ref, cparts_ref, u1a_ref, u1b_ref, bu1_ref,
                u2_ref, bu2_ref, a_ref, ba_ref, b_ref, bb_ref, o_ref):
    agg = _agg_from_parts(parts_ref, cparts_ref)
    hn = _node_mlp(h_ref[...], agg, u1a_ref, u1b_ref, bu1_ref, u2_ref, bu2_ref)
    z = jnp.maximum(
        jnp.dot(hn, a_ref[...], preferred_element_type=_f32) + ba_ref[...],
        0.0)
    o_ref[...] = jnp.dot(z, b_ref[...], preferred_element_type=_f32) \
        + bb_ref[...]


def _final_call(h, parts, cparts, U1a, U1b, bU1l, U2l, bU2l, A, bA, B, bB):
    return pl.pallas_call(
        _final_body,
        out_shape=jax.ShapeDtypeStruct((N, 20), _f32),
    )(h, parts, cparts, U1a, U1b, bU1l, U2l, bU2l, A, bA, B, bB)


def _build_decoder_mats(Cw1, Cb1, Cw2, Cb2):
    k = jnp.arange(15)
    t = jnp.arange(29)
    rows = 4 * t[None, :] + k[:, None]
    tcol = jnp.broadcast_to(t[None, :], (15, 29))
    valsA = jnp.broadcast_to(Cw1[:, 0, :].T[:, None, :], (15, 29, 8))
    A = jnp.zeros((128, 8, 29), _f32).at[rows, :, tcol].set(
        jnp.transpose(valsA, (0, 1, 2)))
    A = A.reshape(128, 8 * 29)
    bA = jnp.broadcast_to(Cb1[:, None], (8, 29)).reshape(1, 8 * 29)
    dt = jnp.arange(10)
    tp = jnp.arange(20)
    tt = tp[None, :] + dt[:, None]
    tpb = jnp.broadcast_to(tp[None, :], (10, 20))
    valsB = jnp.broadcast_to(Cw2[0][:, :, None], (8, 10, 20))
    B = jnp.zeros((8, 29, 20), _f32).at[:, tt, tpb].set(valsB)
    B = B.reshape(8 * 29, 20)
    bB = Cw2.dtype.type(0) + Cb2.reshape(1, 1)
    return A, bA, B, bB


def kernel(x, edge_index, edge_attr, W_emb1, b_emb1, W_emb2, b_emb2,
           M1, bM1, M2, bM2, U1, bU1, U2, bU2, Cw1, Cb1, Cw2, Cb2):
    pad = EPAD - E
    src = edge_index[0]
    dst = edge_index[1]
    izeros = jnp.zeros((pad,), jnp.int32)
    dst_g = jnp.concatenate([dst, izeros])
    src_g = jnp.concatenate([src, izeros])
    dst_s = jnp.concatenate([dst, jnp.full((pad,), NP - 1, jnp.int32)])
    dst3 = dst_s.reshape(NW, NCHUNK, CH)
    ea_pad = jnp.concatenate(
        [edge_attr, jnp.zeros((pad, D_EDGE), _f32)], axis=0)
    cz = jnp.stack([jnp.zeros((CH, H), _f32), jnp.ones((CH, H), _f32)])
    cnt8 = _counts_call(dst3, cz)[:, :, :8]

    h, P, Q = _embed_call(
        x, W_emb1[:H], b_emb1.reshape(1, H), W_emb2, b_emb2.reshape(1, H),
        M1[0, :H], M1[0, H:2 * H])

    out = None
    for l in range(L):
        g = _gather_call(P, Q, dst_g, src_g)
        m = _edge_call(g, ea_pad, M1[l, 2 * H:], bM1[l].reshape(1, H),
                       M2[l], bM2[l].reshape(1, H))
        parts = _scatter_call(m, dst3, cz)
        if l < L - 1:
            h, P, Q = _node_call(
                h, parts, cnt8, U1[l, :H], U1[l, H:], bU1[l].reshape(1, H),
                U2[l], bU2[l].reshape(1, H), M1[l + 1, :H], M1[l + 1, H:2 * H])
        else:
            A, bA, B, bB = _build_decoder_mats(Cw1, Cb1, Cw2, Cb2)
            out = _final_call(
                h, parts, cnt8, U1[l, :H], U1[l, H:], bU1[l].reshape(1, H),
                U2[l], bU2[l].reshape(1, H), A, bA, B, bB)
    return out


# speedup vs baseline: 2.2990x; 1.0011x over previous
import functools

import jax
import jax.numpy as jnp
from jax import lax
from jax.experimental import pallas as pl
from jax.experimental.pallas import tpu as pltpu
from jax.experimental.pallas import tpu_sc as plsc

N = 10000
E = 160000
H = 128
D_EDGE = 16
L = 4

NC = 2
NS = 16
NW = NC * NS
CH = 80
RING = 4
GRING = 2
EPT = 5120
EPAD = NW * EPT
NCHUNK = EPT // CH
NP = 10112
STRIPE = NP // NS
_TCHUNKS = [(i * CH, CH) for i in range(STRIPE // CH)]
if STRIPE % CH:
    _TCHUNKS.append((STRIPE - STRIPE % CH, STRIPE % CH))
RPT = NP // NS
_RCHUNKS = [(i * CH, CH) for i in range(RPT // CH)]
if RPT % CH:
    _RCHUNKS.append((RPT - RPT % CH, RPT % CH))
BE = 2048

_f32 = jnp.float32
_bf16 = jnp.bfloat16


def _mesh():
    return plsc.VectorSubcoreMesh(
        core_axis_name="c", subcore_axis_name="s", num_cores=NC, num_subcores=NS)


def _drain(src, dst, sem):
    pltpu.make_async_copy(src, dst, sem).wait()


def _gather_body(p_hbm, q_hbm, dst_hbm, src_hbm, g_hbm, dall, sall, *bufs):
    c = lax.axis_index("c")
    s = lax.axis_index("s")
    wid = s * NC + c
    base = wid * EPT
    pbuf = bufs[0:RING]
    qbuf = bufs[RING:2 * RING]
    sp = bufs[2 * RING:3 * RING]
    sq = bufs[3 * RING:4 * RING]
    wp = bufs[4 * RING:5 * RING]
    wq = bufs[5 * RING:6 * RING]

    pltpu.sync_copy(dst_hbm.at[pl.ds(base, EPT)], dall)
    pltpu.sync_copy(src_hbm.at[pl.ds(base, EPT)], sall)

    def issue_gather(i, b):
        pltpu.async_copy(p_hbm.at[dall.at[pl.ds(i * CH, CH)]], pbuf[b], sp[b])
        pltpu.async_copy(q_hbm.at[sall.at[pl.ds(i * CH, CH)]], qbuf[b], sq[b])

    def issue_writeout(j, b):
        _drain(p_hbm.at[pl.ds(0, CH)], pbuf[b], sp[b])
        _drain(q_hbm.at[pl.ds(0, CH)], qbuf[b], sq[b])
        off = base + j * CH
        pltpu.async_copy(pbuf[b], g_hbm.at[pl.ds(off, CH), pl.ds(0, H)], wp[b])
        pltpu.async_copy(qbuf[b], g_hbm.at[pl.ds(off, CH), pl.ds(H, H)], wq[b])

    def drain_writeout(b):
        _drain(pbuf[b], g_hbm.at[pl.ds(base, CH), pl.ds(0, H)], wp[b])
        _drain(qbuf[b], g_hbm.at[pl.ds(base, CH), pl.ds(H, H)], wq[b])

    def body(k, carry):
        for b in range(RING):
            i = RING * k + b

            @pl.when(k > 0)
            def _free():
                drain_writeout(b)

            issue_gather(i, b)
            bb = (b + 1) % RING
            j = i - (RING - 1)
            if b == RING - 1:
                issue_writeout(RING * k, bb)
            else:
                @pl.when(k > 0)
                def _wout():
                    issue_writeout(j, bb)
        return carry

    lax.fori_loop(0, NCHUNK // RING, body, 0)
    for t in range(RING - 1):
        j = NCHUNK - (RING - 1) + t
        issue_writeout(j, j % RING)
    for b in range(RING):
        drain_writeout(b)


def _gather_call(P, Q, dst_g, src_g):
    k = pl.kernel(
        _gather_body,
        out_type=jax.ShapeDtypeStruct((EPAD, 2 * H), _f32),
        mesh=_mesh(),
        scratch_types=[
            pltpu.VMEM((EPT,), jnp.int32),
            pltpu.VMEM((EPT,), jnp.int32),
        ] + [pltpu.VMEM((CH, H), _f32)] * (2 * RING)
          + [pltpu.SemaphoreType.DMA] * (4 * RING),
    )
    return k(P, Q, dst_g, src_g)


def _zero_acc_stripes(cz_hbm, mrows, acc, s):
    pltpu.sync_copy(cz_hbm.at[0], mrows)
    for boff, bsz in _RCHUNKS:
        r0 = s * RPT + boff
        pltpu.sync_copy(mrows.at[pl.ds(0, bsz)], acc.at[pl.ds(r0, bsz)])


def _writeout_acc_stripes(parts_hbm, mrows, acc, c, s):
    for boff, bsz in _RCHUNKS:
        r0 = s * RPT + boff
        pltpu.sync_copy(acc.at[pl.ds(r0, bsz)], mrows.at[pl.ds(0, bsz)])
        pltpu.sync_copy(mrows.at[pl.ds(0, bsz)], parts_hbm.at[c, pl.ds(r0, bsz)])


def _scatter_body(m_hbm, dst3_hbm, cz_hbm, parts_hbm, didx2, acc, *bufs):
    c = lax.axis_index("c")
    s = lax.axis_index("s")
    wid = s * NC + c
    mbuf = bufs[0:RING]
    sm = bufs[RING:2 * RING]
    sc = bufs[2 * RING:3 * RING]
    _zero_acc_stripes(cz_hbm, mbuf[0], acc, s)
    pltpu.sync_copy(dst3_hbm.at[wid], didx2)
    plsc.subcore_barrier()

    def issue_load(i, b):
        off = wid * EPT + i * CH
        pltpu.async_copy(m_hbm.at[pl.ds(off, CH)], mbuf[b], sm[b])

    def issue_scatter(j, b):
        _drain(m_hbm.at[pl.ds(0, CH)], mbuf[b], sm[b])
        pltpu.async_copy(mbuf[b], acc.at[didx2.at[j]], sc[b], add=True)

    def drain_scatter(b):
        _drain(mbuf[b], acc.at[didx2.at[0]], sc[b])

    def body(k, carry):
        for b in range(RING):
            i = RING * k + b

            @pl.when(k > 0)
            def _free():
                drain_scatter(b)

            issue_load(i, b)
            bb = (b + 1) % RING
            j = i - (RING - 1)
            if b == RING - 1:
                issue_scatter(RING * k, bb)
            else:
                @pl.when(k > 0)
                def _sc():
                    issue_scatter(j, bb)
        return carry

    lax.fori_loop(0, NCHUNK // RING, body, 0)
    for t in range(RING - 1):
        j = NCHUNK - (RING - 1) + t
        issue_scatter(j, j % RING)
    for b in range(RING):
        drain_scatter(b)
    plsc.subcore_barrier()
    _writeout_acc_stripes(parts_hbm, mbuf[0], acc, c, s)


def _counts_body(dst3_hbm, cz_hbm, parts_hbm, didx2, acc, *bufs):
    c = lax.axis_index("c")
    s = lax.axis_index("s")
    wid = s * NC + c
    zb = bufs[0]
    ones = bufs[1]
    sc = bufs[2 * RING:3 * RING]
    _zero_acc_stripes(cz_hbm, zb, acc, s)
    pltpu.sync_copy(dst3_hbm.at[wid], didx2)
    pltpu.sync_copy(cz_hbm.at[1], ones)
    plsc.subcore_barrier()

    def body(k, carry):
        for b in range(RING):
            i = RING * k + b

            @pl.when(k > 0)
            def _free():
                _drain(ones, acc.at[didx2.at[0]], sc[b])

            pltpu.async_copy(ones, acc.at[didx2.at[i]], sc[b], add=True)
        return carry

    lax.fori_loop(0, NCHUNK // RING, body, 0)
    for b in range(RING):
        _drain(ones, acc.at[didx2.at[0]], sc[b])
    plsc.subcore_barrier()
    _writeout_acc_stripes(parts_hbm, zb, acc, c, s)


_SC_SCRATCH = [
    pltpu.VMEM((NCHUNK, CH), jnp.int32),
    pltpu.VMEM_SHARED((NP, H), _f32),
] + [pltpu.VMEM((CH, H), _f32)] * RING \
  + [pltpu.SemaphoreType.DMA] * (2 * RING)


def _scatter_call(m, dst3, cz):
    k = pl.kernel(
        _scatter_body,
        out_type=jax.ShapeDtypeStruct((NC, NP, H), _f32),
        mesh=_mesh(),
        scratch_types=_SC_SCRATCH,
    )
    return k(m, dst3, cz)


def _counts_call(dst3, cz):
    k = pl.kernel(
        _counts_body,
        out_type=jax.ShapeDtypeStruct((NC, NP, H), _f32),
        mesh=_mesh(),
        scratch_types=_SC_SCRATCH,
    )
    return k(dst3, cz)


def _pack_cols(x):
    lo = x[:, :64].astype(_bf16).astype(_f32)
    hi = x[:, 64:].astype(_bf16).astype(_f32)
    lo_b = jax.lax.bitcast_convert_type(lo, jnp.uint32) >> 16
    hi_b = jax.lax.bitcast_convert_type(hi, jnp.uint32) \
        & jnp.uint32(0xFFFF0000)
    return jax.lax.bitcast_convert_type(lo_b | hi_b, jnp.int32)


def _unpack_cols(g):
    u = jax.lax.bitcast_convert_type(g, jnp.uint32)
    lo = jax.lax.bitcast_convert_type(u << 16, _f32)
    hi = jax.lax.bitcast_convert_type(u & jnp.uint32(0xFFFF0000), _f32)
    return lo, hi


def _embed_body(x_ref, w1_ref, b1_ref, w2_ref, b2_ref, ma_ref, mb_ref,
                h_ref, p_ref, q_ref):
    h = jnp.maximum(
        jnp.dot(x_ref[...], w1_ref[...], preferred_element_type=_f32)
        + b1_ref[...], 0.0)
    h = jnp.maximum(
        jnp.dot(h, w2_ref[...], preferred_element_type=_f32)
        + b2_ref[...], 0.0)
    h_ref[...] = h
    p_ref[...] = jnp.dot(h, ma_ref[...], preferred_element_type=_f32)
    q_ref[...] = jnp.dot(h, mb_ref[...], preferred_element_type=_f32)


def _embed_call(x, W1, b1, W2, b2, Ma, Mb):
    return pl.pallas_call(
        _embed_body,
        out_shape=(
            jax.ShapeDtypeStruct((N, H), _f32),
            jax.ShapeDtypeStruct((N, H), _f32),
            jax.ShapeDtypeStruct((N, H), _f32),
        ),
    )(x, W1, b1, W2, b2, Ma, Mb)


def _edge_body(g_ref, ea_ref, m1c_ref, bm1_ref, m2_ref, bm2_ref, o_ref):
    r = jnp.dot(ea_ref[...], m1c_ref[...], preferred_element_type=_f32) \
        + bm1_ref[...]
    t = jnp.maximum(g_ref[:, :H] + g_ref[:, H:] + r, 0.0)
    o_ref[...] = jnp.maximum(
        jnp.dot(t, m2_ref[...], preferred_element_type=_f32) + bm2_ref[...],
        0.0)


def _edge_call(g, ea_pad, M1c, bM1l, M2l, bM2l):
    return pl.pallas_call(
        _edge_body,
        grid=(EPAD // BE,),
        in_specs=[
            pl.BlockSpec((BE, 2 * H), lambda i: (i, 0)),
            pl.BlockSpec((BE, D_EDGE), lambda i: (i, 0)),
            pl.BlockSpec((D_EDGE, H), lambda i: (0, 0)),
            pl.BlockSpec((1, H), lambda i: (0, 0)),
            pl.BlockSpec((H, H), lambda i: (0, 0)),
            pl.BlockSpec((1, H), lambda i: (0, 0)),
        ],
        out_specs=pl.BlockSpec((BE, H), lambda i: (i, 0)),
        out_shape=jax.ShapeDtypeStruct((EPAD, H), _f32),
    )(g, ea_pad, M1c, bM1l, M2l, bM2l)


def _agg_from_parts(parts_ref, cparts_ref):
    counts = cparts_ref[0, :N, 0] + cparts_ref[1, :N, 0]
    inv = 1.0 / jnp.maximum(counts, 1.0)
    agg = (parts_ref[0, :N, :] + parts_ref[1, :N, :]) * inv[:, None]
    return agg


def _node_mlp(h, agg, u1a_ref, u1b_ref, bu1_ref, u2_ref, bu2_ref):
    u = jnp.maximum(
        jnp.dot(h, u1a_ref[...], preferred_element_type=_f32)
        + jnp.dot(agg, u1b_ref[...], preferred_element_type=_f32)
        + bu1_ref[...], 0.0)
    u = jnp.maximum(
        jnp.dot(u, u2_ref[...], preferred_element_type=_f32)
        + bu2_ref[...], 0.0)
    return h + u


def _node_body(h_ref, parts_ref, cparts_ref, u1a_ref, u1b_ref, bu1_ref,
               u2_ref, bu2_ref, ma_ref, mb_ref, h_out, p_out, q_out):
    agg = _agg_from_parts(parts_ref, cparts_ref)
    hn = _node_mlp(h_ref[...], agg, u1a_ref, u1b_ref, bu1_ref, u2_ref, bu2_ref)
    h_out[...] = hn
    p_out[...] = jnp.dot(hn, ma_ref[...], preferred_element_type=_f32)
    q_out[...] = jnp.dot(hn, mb_ref[...], preferred_element_type=_f32)


def _node_call(h, parts, cparts, U1a, U1b, bU1l, U2l, bU2l, Ma, Mb):
    return pl.pallas_call(
        _node_body,
        out_shape=(
            jax.ShapeDtypeStruct((N, H), _f32),
            jax.ShapeDtypeStruct((N, H), _f32),
            jax.ShapeDtypeStruct((N, H), _f32),
        ),
    )(h, parts, cparts, U1a, U1b, bU1l, U2l, bU2l, Ma, Mb)


def _final_body(h_ref, parts_ref, cparts_ref, u1a_ref, u1b_ref, bu1_ref,
                u2_ref, bu2_ref, a_ref, ba_ref, b_ref, bb_ref, o_ref):
    agg = _agg_from_parts(parts_ref, cparts_ref)
    hn = _node_mlp(h_ref[...], agg, u1a_ref, u1b_ref, bu1_ref, u2_ref, bu2_ref)
    z = jnp.maximum(
        jnp.dot(hn, a_ref[...], preferred_element_type=_f32) + ba_ref[...],
        0.0)
    o_ref[...] = jnp.dot(z, b_ref[...], preferred_element_type=_f32) \
        + bb_ref[...]


def _final_call(h, parts, cparts, U1a, U1b, bU1l, U2l, bU2l, A, bA, B, bB):
    return pl.pallas_call(
        _final_body,
        out_shape=jax.ShapeDtypeStruct((N, 20), _f32),
    )(h, parts, cparts, U1a, U1b, bU1l, U2l, bU2l, A, bA, B, bB)


def _build_decoder_mats(Cw1, Cb1, Cw2, Cb2):
    k = jnp.arange(15)
    t = jnp.arange(29)
    rows = 4 * t[None, :] + k[:, None]
    tcol = jnp.broadcast_to(t[None, :], (15, 29))
    valsA = jnp.broadcast_to(Cw1[:, 0, :].T[:, None, :], (15, 29, 8))
    A = jnp.zeros((128, 8, 29), _f32).at[rows, :, tcol].set(
        jnp.transpose(valsA, (0, 1, 2)))
    A = A.reshape(128, 8 * 29)
    bA = jnp.broadcast_to(Cb1[:, None], (8, 29)).reshape(1, 8 * 29)
    dt = jnp.arange(10)
    tp = jnp.arange(20)
    tt = tp[None, :] + dt[:, None]
    tpb = jnp.broadcast_to(tp[None, :], (10, 20))
    valsB = jnp.broadcast_to(Cw2[0][:, :, None], (8, 10, 20))
    B = jnp.zeros((8, 29, 20), _f32).at[:, tt, tpb].set(valsB)
    B = B.reshape(8 * 29, 20)
    bB = Cw2.dtype.type(0) + Cb2.reshape(1, 1)
    return A, bA, B, bB


def kernel(x, edge_index, edge_attr, W_emb1, b_emb1, W_emb2, b_emb2,
           M1, bM1, M2, bM2, U1, bU1, U2, bU2, Cw1, Cb1, Cw2, Cb2):
    pad = EPAD - E
    src = edge_index[0]
    dst = edge_index[1]
    izeros = jnp.zeros((pad,), jnp.int32)
    dst_g = jnp.concatenate([dst, izeros])
    src_g = jnp.concatenate([src, izeros])
    dst_s = jnp.concatenate([dst, jnp.full((pad,), NP - 1, jnp.int32)])
    dst3 = dst_s.reshape(NW, NCHUNK, CH)
    ea_pad = jnp.concatenate(
        [edge_attr, jnp.zeros((pad, D_EDGE), _f32)], axis=0)
    cz = jnp.stack([jnp.zeros((CH, H), _f32), jnp.ones((CH, H), _f32)])
    cnt8 = _counts_call(dst3, cz)[:, :, :8]

    h, P, Q = _embed_call(
        x, W_emb1[:H], b_emb1.reshape(1, H), W_emb2, b_emb2.reshape(1, H),
        M1[0, :H], M1[0, H:2 * H])

    out = None
    for l in range(L):
        g = _gather_call(P, Q, dst_g, src_g)
        m = _edge_call(g, ea_pad, M1[l, 2 * H:], bM1[l].reshape(1, H),
                       M2[l], bM2[l].reshape(1, H))
        parts = _scatter_call(m, dst3, cz)
        if l < L - 1:
            h, P, Q = _node_call(
                h, parts, cnt8, U1[l, :H], U1[l, H:], bU1[l].reshape(1, H),
                U2[l], bU2[l].reshape(1, H), M1[l + 1, :H], M1[l + 1, H:2 * H])
        else:
            A, bA, B, bB = _build_decoder_mats(Cw1, Cb1, Cw2, Cb2)
            out = _final_call(
                h, parts, cnt8, U1[l, :H], U1[l, H:], bU1[l].reshape(1, H),
                U2[l], bU2[l].reshape(1, H), A, bA, B, bB)
    return out
